# baseline (speedup 1.0000x reference)
import jax
import jax.numpy as jnp
from jax import lax
from jax.experimental import pallas as pl
from jax.experimental.pallas import tpu as pltpu
from jax.experimental.pallas import tpu_sc as plsc

N_NODES = 10000
N_EDGES = 320000
D = 128

NUM_CORES = 2
NUM_SUBCORES = 16
NW = NUM_CORES * NUM_SUBCORES
RANGE = 313
SHARD = 320
SEG = 2560
N_SEG = N_EDGES // SEG
V_PER_SEG = SEG // 16
CHUNK = 128
LISTCAP = 2816


def _np_kernel(x_hbm, idx_hbm, out_hbm, shard, iseg0, iseg1, plist, llist,
               pidx0, pidx1, stag0, stag1, isem0, isem1, wsem0, wsem1, ssem):
    isegs = (iseg0, iseg1)
    isems = (isem0, isem1)
    pidxs = (pidx0, pidx1)
    stags = (stag0, stag1)
    wsems = (wsem0, wsem1)

    wid = lax.axis_index("s") * NUM_CORES + lax.axis_index("c")
    lo = wid * RANGE
    hi = lo + RANGE
    stage_base = pl.multiple_of(jnp.minimum(lo - lo % 8, N_NODES - SHARD), 8)

    iota16 = lax.iota(jnp.int32, 16)
    col_idx = [iota16 + 16 * c for c in range(8)]

    pltpu.async_copy(x_hbm.at[pl.ds(stage_base, SHARD)], shard, ssem)
    pltpu.make_async_copy(x_hbm.at[pl.ds(0, SHARD)], shard, ssem).wait()

    pltpu.async_copy(idx_hbm.at[pl.ds(0, SEG)], iseg0, isem0)

    def drain_chunk(src_off, cc):
        for r in range(2):
            @pl.when(cc % 2 == r)
            def _():
                @pl.when(cc >= 2)
                def _():
                    pltpu.make_async_copy(
                        stags[r], out_hbm.at[pidxs[r]], wsems[r]).wait()
                for k in range(8):
                    pidxs[r][pl.ds(16 * k, 16)] = (
                        plist[pl.ds(src_off + 16 * k, 16)])

                def rep(grp, c2):
                    for j in range(16):
                        e = src_off + grp * 16 + j
                        locb = plsc.load_gather(
                            llist, [jnp.full((16,), e, jnp.int32)])
                        for c in range(8):
                            stags[r][grp * 16 + j, pl.ds(16 * c, 16)] = (
                                plsc.load_gather(shard, [locb, col_idx[c]]))
                    return c2

                lax.fori_loop(0, 8, rep, 0)
                pltpu.async_copy(stags[r], out_hbm.at[pidxs[r]], wsems[r])
        return cc + 1

    def scan_segment(s, p, cur_s):
        pltpu.make_async_copy(
            idx_hbm.at[pl.ds(0, SEG)], isegs[p], isems[p]).wait()

        def scan_body(k, cur_v):
            v = isegs[p][pl.ds(16 * k, 16)]
            m = jnp.logical_and(v >= lo, v < hi)
            mi = jnp.where(m, 1, 0)
            pos = iota16 + (s * SEG + 16 * k)
            loc = v - stage_base
            tgt = cur_v + lax.cumsum(mi, axis=0) - 1
            plsc.store_scatter(plist, [tgt], pos, mask=m)
            plsc.store_scatter(llist, [tgt], loc, mask=m)
            return cur_v + plsc.all_reduce_population_count(m)

        cur_v = lax.fori_loop(
            0, V_PER_SEG, scan_body, jnp.full((16,), cur_s, jnp.int32))
        return lax.reduce_max(cur_v, axes=(0,))

    def drain_and_compact(cur_s, cc):
        nch = cur_s // CHUNK

        def dbody(k, cc2):
            return drain_chunk(k * CHUNK, cc2)

        cc = lax.fori_loop(0, nch, dbody, cc)
        rem_off = nch * CHUNK

        @pl.when(rem_off > 0)
        def _():
            for k in range(8):
                plist[pl.ds(16 * k, 16)] = plist[pl.ds(rem_off + 16 * k, 16)]
                llist[pl.ds(16 * k, 16)] = llist[pl.ds(rem_off + 16 * k, 16)]

        return cur_s - rem_off, cc

    def seg_pair(j, carry):
        cur_s, cc = carry
        for p in range(2):
            s = 2 * j + p
            pltpu.async_copy(
                idx_hbm.at[pl.ds((s + 1) * SEG, SEG)], isegs[1 - p],
                isems[1 - p])
            cur_s = scan_segment(s, p, cur_s)
            cur_s, cc = drain_and_compact(cur_s, cc)
        return cur_s, cc

    cur_s, cc = lax.fori_loop(0, (N_SEG - 1) // 2, seg_pair, (0, 0))

    cur_s = scan_segment(N_SEG - 1, 0, cur_s)
    cur_s, cc = drain_and_compact(cur_s, cc)

    @pl.when(cur_s > 0)
    def _():
        zeros = jnp.zeros((16,), jnp.int32)
        p0 = plsc.load_gather(plist, [zeros])
        l0 = plsc.load_gather(llist, [zeros])
        for k in range(8):
            plist[pl.ds(cur_s + 16 * k, 16)] = p0
            llist[pl.ds(cur_s + 16 * k, 16)] = l0
        drain_chunk(0, cc)

    cc = cc + jnp.where(cur_s > 0, 1, 0)

    for r in range(2):
        @pl.when(jnp.logical_and(cc >= 1, (cc - 1) % 2 == r))
        def _():
            pltpu.make_async_copy(
                stags[r], out_hbm.at[pidxs[r]], wsems[r]).wait()

        @pl.when(jnp.logical_and(cc >= 2, (cc - 2) % 2 == r))
        def _():
            pltpu.make_async_copy(
                stags[r], out_hbm.at[pidxs[r]], wsems[r]).wait()


@jax.jit
def _gather(x, idx):
    mesh = plsc.VectorSubcoreMesh(core_axis_name="c", subcore_axis_name="s")
    return pl.kernel(
        _np_kernel,
        out_type=jax.ShapeDtypeStruct((N_EDGES, D), jnp.float32),
        mesh=mesh,
        compiler_params=pltpu.CompilerParams(needs_layout_passes=False),
        scratch_types=[
            pltpu.VMEM((SHARD, D), jnp.float32),
            pltpu.VMEM((SEG,), jnp.int32),
            pltpu.VMEM((SEG,), jnp.int32),
            pltpu.VMEM((LISTCAP,), jnp.int32),
            pltpu.VMEM((LISTCAP,), jnp.int32),
            pltpu.VMEM((CHUNK,), jnp.int32),
            pltpu.VMEM((CHUNK,), jnp.int32),
            pltpu.VMEM((CHUNK, D), jnp.float32),
            pltpu.VMEM((CHUNK, D), jnp.float32),
            pltpu.SemaphoreType.DMA,
            pltpu.SemaphoreType.DMA,
            pltpu.SemaphoreType.DMA,
            pltpu.SemaphoreType.DMA,
            pltpu.SemaphoreType.DMA,
        ],
    )(x, idx)


def kernel(x, edge_index):
    return _gather(x, edge_index.astype(jnp.int32).reshape(-1))


# speedup vs baseline: 4.7479x; 4.7479x over previous
import jax
import jax.numpy as jnp
from jax import lax
from jax.experimental import pallas as pl
from jax.experimental.pallas import tpu as pltpu
from jax.experimental.pallas import tpu_sc as plsc

N_NODES = 10000
N_EDGES = 320000
D = 128

NUM_CORES = 2
NUM_SUBCORES = 16
NW = NUM_CORES * NUM_SUBCORES
B_PER_W = N_EDGES // NW
CHUNK = 200
N_FULL = B_PER_W // CHUNK
NBUF = 2


def _gather_kernel(x_hbm, idx_hbm, out_hbm, idx_v, buf0, buf1, shm,
                   gsem0, gsem1, wsem0, wsem1, dsem0, dsem1):
    bufs = (buf0, buf1)
    gsems = (gsem0, gsem1)
    wsems = (wsem0, wsem1)
    dsems = (dsem0, dsem1)

    sid = lax.axis_index("s")
    wid = sid * NUM_CORES + lax.axis_index("c")
    base = wid * B_PER_W

    pltpu.sync_copy(idx_hbm.at[pl.ds(base, B_PER_W)], idx_v)

    def start_gather(g, b):
        pltpu.async_copy(
            x_hbm.at[idx_v.at[pl.ds(g * CHUNK, CHUNK)]], bufs[b], gsems[b])

    def wait_gather(b):
        pltpu.make_async_copy(
            x_hbm.at[idx_v.at[pl.ds(0, CHUNK)]], bufs[b], gsems[b]).wait()

    def start_write(g, b):
        pltpu.async_copy(
            bufs[b], out_hbm.at[pl.ds(base + g * CHUNK, CHUNK)], wsems[b])

    def start_relay_write(b, t):
        pltpu.async_copy(bufs[b], shm.at[sid, t], wsems[b])

    def wait_write(b):
        pltpu.make_async_copy(
            bufs[b], out_hbm.at[pl.ds(base, CHUNK)], wsems[b]).wait()

    def start_dma(g, t):
        pltpu.async_copy(
            shm.at[sid, t], out_hbm.at[pl.ds(base + g * CHUNK, CHUNK)],
            dsems[t])

    def wait_dma(t):
        pltpu.make_async_copy(
            shm.at[sid, t], out_hbm.at[pl.ds(base, CHUNK)], dsems[t]).wait()

    start_gather(0, 0)

    def body(j, carry):
        for b in range(NBUF):
            g = j * 2 + b
            bn = (b + 1) % NBUF

            @pl.when(g + 1 < N_FULL)
            def _():
                @pl.when(g >= 1)
                def _():
                    wait_write(bn)
                if b == 0:
                    @pl.when(j >= 1)
                    def _():
                        for t in range(2):
                            @pl.when((j - 1) % 2 == t)
                            def _():
                                start_dma(g - 1, t)
                start_gather(g + 1, bn)

            wait_gather(b)
            if b == 0:
                start_write(g, b)
            else:
                for t in range(2):
                    @pl.when(j % 2 == t)
                    def _():
                        @pl.when(j >= 2)
                        def _():
                            wait_dma(t)
                        start_relay_write(b, t)
        return carry

    lax.fori_loop(0, N_FULL // 2, body, 0)

    wait_write(0)
    wait_write(1)
    start_dma(N_FULL - 1, 0)
    wait_dma(1)
    wait_dma(0)


@jax.jit
def _gather(x, idx):
    mesh = plsc.VectorSubcoreMesh(core_axis_name="c", subcore_axis_name="s")
    return pl.kernel(
        _gather_kernel,
        out_type=jax.ShapeDtypeStruct((N_EDGES, D), jnp.float32),
        mesh=mesh,
        scratch_types=[
            pltpu.VMEM((B_PER_W,), jnp.int32),
            pltpu.VMEM((CHUNK, D), jnp.float32),
            pltpu.VMEM((CHUNK, D), jnp.float32),
            pltpu.VMEM_SHARED((NUM_SUBCORES, 2, CHUNK, D), jnp.float32),
            pltpu.SemaphoreType.DMA,
            pltpu.SemaphoreType.DMA,
            pltpu.SemaphoreType.DMA,
            pltpu.SemaphoreType.DMA,
            pltpu.SemaphoreType.DMA,
            pltpu.SemaphoreType.DMA,
        ],
    )(x, idx)


def kernel(x, edge_index):
    return _gather(x, edge_index.astype(jnp.int32).reshape(-1))
